# baseline (speedup 1.0000x reference)
import functools
import jax
import jax.numpy as jnp
from jax import lax
from jax.experimental import pallas as pl
from jax.experimental.pallas import tpu as pltpu
from jax.experimental.pallas import tpu_sc as plsc

N_GRAPHS = 64
RATIO = 0.8
EPS = 1e-5

_NC, _NS = 2, 16
_NP = 10240
_TRASH = 10000
_E = 320000
_EPT = _E // (_NC * _NS)
_CE = 128
_NFULL = _EPT // _CE
_CT = _EPT - _NFULL * _CE


def _agg_sc_body(h_hbm, src_hbm, dst_hbm, z_hbm, out_hbm,
                 is_v, id_v, rows_v, ist_v, idt_v, rowst_v, acc_sh, sem):
    c = lax.axis_index("c")
    s = lax.axis_index("s")
    rpt = _NP // _NS
    r0 = s * rpt
    pltpu.sync_copy(z_hbm.at[pl.ds(r0, rpt)], acc_sh.at[pl.ds(r0, rpt)])
    plsc.subcore_barrier()
    base = (c * _NS + s) * _EPT

    def chunk(i, carry):
        off = base + i * _CE
        pltpu.sync_copy(src_hbm.at[pl.ds(off, _CE)], is_v)
        pltpu.sync_copy(dst_hbm.at[pl.ds(off, _CE)], id_v)
        pltpu.async_copy(h_hbm.at[is_v], rows_v, sem).wait()
        pltpu.sync_copy(rows_v, acc_sh.at[id_v], add=True)
        return carry

    lax.fori_loop(0, _NFULL, chunk, 0)
    offt = base + _NFULL * _CE
    pltpu.sync_copy(src_hbm.at[pl.ds(offt, _CT)], ist_v)
    pltpu.sync_copy(dst_hbm.at[pl.ds(offt, _CT)], idt_v)
    pltpu.async_copy(h_hbm.at[ist_v], rowst_v, sem).wait()
    pltpu.sync_copy(rowst_v, acc_sh.at[idt_v], add=True)
    plsc.subcore_barrier()
    pltpu.sync_copy(acc_sh.at[pl.ds(r0, rpt)],
                    out_hbm.at[c].at[pl.ds(r0, rpt)])


def _make_agg():
    mesh = plsc.VectorSubcoreMesh(core_axis_name="c", subcore_axis_name="s")
    return functools.partial(
        pl.kernel, mesh=mesh,
        out_type=jax.ShapeDtypeStruct((_NC, _NP, 128), jnp.float32),
        scratch_types=[
            pltpu.VMEM((_CE,), jnp.int32),
            pltpu.VMEM((_CE,), jnp.int32),
            pltpu.VMEM((_CE, 128), jnp.float32),
            pltpu.VMEM((_CT,), jnp.int32),
            pltpu.VMEM((_CT,), jnp.int32),
            pltpu.VMEM((_CT, 128), jnp.float32),
            pltpu.VMEM_SHARED((_NP, 128), jnp.float32),
            pltpu.SemaphoreType.DMA,
        ])(_agg_sc_body)


def _agg_rows(table, src, dst):
    zeros = jnp.zeros((_NP, 128), jnp.float32)
    return _make_agg()(table, src, dst, zeros)


def _mm_scale_body(x_ref, w_ref, dis_ref, o_ref):
    o_ref[...] = jnp.dot(x_ref[...], w_ref[...],
                         preferred_element_type=jnp.float32) * dis_ref[...]


def _mm_scale(x, W, dis_col):
    BR = 1024
    return pl.pallas_call(
        _mm_scale_body,
        grid=(_NP // BR,),
        in_specs=[pl.BlockSpec((BR, 128), lambda i: (i, 0)),
                  pl.BlockSpec((128, 128), lambda i: (0, 0)),
                  pl.BlockSpec((BR, 1), lambda i: (i, 0))],
        out_specs=pl.BlockSpec((BR, 128), lambda i: (i, 0)),
        out_shape=jax.ShapeDtypeStruct((_NP, 128), jnp.float32),
    )(x, W, dis_col)


def _layer_mid_body(a0_ref, a1_ref, hp_ref, dis_ref, b_ref, w_ref,
                    x1_ref, h2_ref):
    agg = a0_ref[...] + a1_ref[...] + hp_ref[...]
    x1 = jnp.maximum(dis_ref[...] * agg + b_ref[...], 0.0)
    x1_ref[...] = x1
    h2_ref[...] = jnp.dot(x1, w_ref[...],
                          preferred_element_type=jnp.float32) * dis_ref[...]


def _layer_mid(a0, a1, hp, dis_col, b, W):
    BR = 1024
    return pl.pallas_call(
        _layer_mid_body,
        grid=(_NP // BR,),
        in_specs=[pl.BlockSpec((BR, 128), lambda i: (i, 0)),
                  pl.BlockSpec((BR, 128), lambda i: (i, 0)),
                  pl.BlockSpec((BR, 128), lambda i: (i, 0)),
                  pl.BlockSpec((BR, 1), lambda i: (i, 0)),
                  pl.BlockSpec((1, 128), lambda i: (0, 0)),
                  pl.BlockSpec((128, 128), lambda i: (0, 0))],
        out_specs=[pl.BlockSpec((BR, 128), lambda i: (i, 0)),
                   pl.BlockSpec((BR, 128), lambda i: (i, 0))],
        out_shape=[jax.ShapeDtypeStruct((_NP, 128), jnp.float32),
                   jax.ShapeDtypeStruct((_NP, 128), jnp.float32)],
    )(a0, a1, hp, dis_col, b, W)


def _blk_out_body(a0_ref, a1_ref, hp_ref, dis_ref, b_ref, x1_ref,
                  w1_ref, w2_ref, lb_ref, o_ref):
    x2 = jnp.maximum(
        dis_ref[...] * (a0_ref[...] + a1_ref[...] + hp_ref[...]) + b_ref[...],
        0.0)
    o_ref[...] = jnp.maximum(
        jnp.dot(x1_ref[...], w1_ref[...], preferred_element_type=jnp.float32)
        + jnp.dot(x2, w2_ref[...], preferred_element_type=jnp.float32)
        + lb_ref[...], 0.0)


def _blk_out(a0, a1, hp, dis_col, b, x1, linW, linb):
    BR = 1024
    return pl.pallas_call(
        _blk_out_body,
        grid=(_NP // BR,),
        in_specs=[pl.BlockSpec((BR, 128), lambda i: (i, 0)),
                  pl.BlockSpec((BR, 128), lambda i: (i, 0)),
                  pl.BlockSpec((BR, 128), lambda i: (i, 0)),
                  pl.BlockSpec((BR, 1), lambda i: (i, 0)),
                  pl.BlockSpec((1, 128), lambda i: (0, 0)),
                  pl.BlockSpec((BR, 128), lambda i: (i, 0)),
                  pl.BlockSpec((128, 128), lambda i: (0, 0)),
                  pl.BlockSpec((128, 128), lambda i: (0, 0)),
                  pl.BlockSpec((1, 128), lambda i: (0, 0))],
        out_specs=pl.BlockSpec((BR, 128), lambda i: (i, 0)),
        out_shape=jax.ShapeDtypeStruct((_NP, 128), jnp.float32),
    )(a0, a1, hp, dis_col, b, x1, linW[:128], linW[128:], linb)


_DEG_TABLE_ROWS = 8


def _block(x, src, dst, emask, W0, b0, W1, b1, linW, linb):
    n = x.shape[0]
    table = jnp.zeros((_DEG_TABLE_ROWS, 128), jnp.float32).at[1].set(1.0)
    eidx = emask.astype(jnp.int32)
    dparts = _agg_rows(table, eidx, dst)
    deg = 1.0 + (dparts[0, :n, 0] + dparts[1, :n, 0])
    dis = deg ** -0.5
    dis_col = jnp.zeros((_NP, 1), jnp.float32).at[:n, 0].set(dis)
    dstm = jnp.where(emask, dst, _TRASH).astype(jnp.int32)

    xpad = jnp.zeros((_NP, 128), jnp.float32).at[:n].set(x)
    h1 = _mm_scale(xpad, W0, dis_col)
    a = _agg_rows(h1, src, dstm)
    x1, h2 = _layer_mid(a[0], a[1], h1, dis_col, b0.reshape(1, 128), W1)
    a2 = _agg_rows(h2, src, dstm)
    out = _blk_out(a2[0], a2[1], h2, dis_col, b1.reshape(1, 128), x1,
                   linW, linb.reshape(1, 128))
    return out[:n]


def _counts_body(b_ref, v_ref, ratio_ref, k_ref):
    Np = b_ref.shape[1]
    G = k_ref.shape[0]
    gi = lax.broadcasted_iota(jnp.int32, (G, 1), 0).astype(jnp.float32)
    acc = jnp.zeros((G, 1), jnp.float32)
    BJ = 2048
    for c in range(Np // BJ):
        bj = b_ref[:, pl.ds(c * BJ, BJ)]
        vj = v_ref[:, pl.ds(c * BJ, BJ)]
        oh = jnp.where(bj == gi, vj, 0.0)
        acc = acc + jnp.sum(oh, axis=1, keepdims=True)
    k_ref[...] = jnp.ceil(ratio_ref[0, 0] * acc)


def _rank_body(sc_ref, bc_ref, vc_ref, sr_ref, br_ref, vr_ref,
               k_ref, koff_ref, tgt_ref, kept_ref, *, n_sentinel):
    BI = sc_ref.shape[0]
    Np = sr_ref.shape[1]
    G = k_ref.shape[1]
    i0 = pl.program_id(0) * BI
    si = sc_ref[...]
    bi = bc_ref[...]
    vi = vc_ref[...]
    ii = i0 + lax.broadcasted_iota(jnp.int32, (BI, 1), 0).astype(jnp.float32)
    acc = jnp.zeros((BI, 1), jnp.float32)
    BJ = 2048
    for c in range(Np // BJ):
        sj = sr_ref[:, pl.ds(c * BJ, BJ)]
        bj = br_ref[:, pl.ds(c * BJ, BJ)]
        vj = vr_ref[:, pl.ds(c * BJ, BJ)]
        jj = c * BJ + lax.broadcasted_iota(jnp.int32, (1, BJ), 1).astype(jnp.float32)
        beats = (sj > si) | ((sj == si) & (jj < ii))
        cmp = (bj == bi) & (vj > 0.0) & beats
        acc = acc + jnp.sum(cmp.astype(jnp.float32), axis=1, keepdims=True)
    ohm = bi == lax.broadcasted_iota(jnp.int32, (1, G), 1).astype(jnp.float32)
    kb = jnp.sum(jnp.where(ohm, k_ref[...], 0.0), axis=1, keepdims=True)
    kob = jnp.sum(jnp.where(ohm, koff_ref[...], 0.0), axis=1, keepdims=True)
    kept = (vi > 0.0) & (acc < kb)
    tgt_ref[...] = jnp.where(kept, kob + acc, float(n_sentinel))
    kept_ref[...] = kept.astype(jnp.float32)


def _topk_pool(x, edge_index, edge_mask, batch, node_mask, p, ratio, num_graphs):
    n, D = x.shape
    G = num_graphs
    Np = ((n + 2047) // 2048) * 2048
    bpad = jnp.zeros((Np, 1), jnp.float32).at[:n, 0].set(batch.astype(jnp.float32))
    vpad = jnp.zeros((Np, 1), jnp.float32).at[:n, 0].set(node_mask.astype(jnp.float32))

    scr = jnp.tanh((x @ p) / jnp.linalg.norm(p))
    scr_col = jnp.zeros((Np, 1), jnp.float32).at[:n, 0].set(scr)

    b_row = bpad.reshape(1, Np)
    v_row = vpad.reshape(1, Np)
    s_row = scr_col.reshape(1, Np)
    ratio_arr = jnp.full((1, 1), ratio, jnp.float32)

    k_col = pl.pallas_call(
        _counts_body,
        out_shape=jax.ShapeDtypeStruct((G, 1), jnp.float32),
    )(b_row, v_row, ratio_arr)
    k_row = k_col.reshape(1, G)
    koff_full = jnp.concatenate([jnp.zeros((1,), jnp.float32),
                                 jnp.cumsum(k_col[:, 0])])
    koff_row = koff_full[:G].reshape(1, G)
    total_kept = koff_full[G].astype(jnp.int32)

    BI = 256
    tgt_col, kept_col = pl.pallas_call(
        functools.partial(_rank_body, n_sentinel=n),
        grid=(Np // BI,),
        in_specs=[pl.BlockSpec((BI, 1), lambda i: (i, 0)),
                  pl.BlockSpec((BI, 1), lambda i: (i, 0)),
                  pl.BlockSpec((BI, 1), lambda i: (i, 0)),
                  pl.BlockSpec((1, Np), lambda i: (0, 0)),
                  pl.BlockSpec((1, Np), lambda i: (0, 0)),
                  pl.BlockSpec((1, Np), lambda i: (0, 0)),
                  pl.BlockSpec((1, G), lambda i: (0, 0)),
                  pl.BlockSpec((1, G), lambda i: (0, 0))],
        out_specs=[pl.BlockSpec((BI, 1), lambda i: (i, 0)),
                   pl.BlockSpec((BI, 1), lambda i: (i, 0))],
        out_shape=[jax.ShapeDtypeStruct((Np, 1), jnp.float32),
                   jax.ShapeDtypeStruct((Np, 1), jnp.float32)],
    )(scr_col, bpad, vpad, s_row, b_row, v_row, k_row, koff_row)

    tgt = tgt_col[:n, 0].astype(jnp.int32)
    kept = kept_col[:n, 0] > 0.0

    xn = jnp.zeros((n + 1, D), x.dtype).at[tgt].set(x * scr[:, None])[:n]
    bnew = jnp.zeros((n + 1,), batch.dtype).at[tgt].set(batch)[:n]
    nmask_new = jnp.arange(n, dtype=jnp.int32) < total_kept
    s, d = edge_index[0], edge_index[1]
    em = edge_mask & kept[s] & kept[d]
    src_new = jnp.where(em, tgt[s], 0).astype(s.dtype)
    dst_new = jnp.where(em, tgt[d], 0).astype(d.dtype)
    einew = jnp.stack([src_new, dst_new])
    return xn, einew, bnew, nmask_new, em


def _gadd(x, b, nmask, G):
    seg = jnp.where(nmask, b, G)
    return jnp.zeros((G + 1, x.shape[1]), x.dtype).at[seg].add(x)[:G]


def _gmax(x, b, nmask, G):
    seg = jnp.where(nmask, b, G)
    return jax.ops.segment_max(x, seg, num_segments=G + 1)[:G]


def _mlp_body(h_ref, g_ref, be_ref, w1_ref, b1_ref, w2_ref, b2_ref, o_ref):
    h = h_ref[...]
    h = (h / jnp.sqrt(1.0 + EPS)) * g_ref[...] + be_ref[...]
    h = jnp.maximum(jnp.dot(h, w1_ref[...], preferred_element_type=jnp.float32)
                    + b1_ref[...], 0.0)
    z = jnp.dot(h, w2_ref[...], preferred_element_type=jnp.float32) + b2_ref[...]
    z = z - jnp.max(z, axis=-1, keepdims=True)
    e = jnp.exp(z)
    o_ref[...] = e / jnp.sum(e, axis=-1, keepdims=True)


def _mlp_head(h, bn_gamma, bn_beta, lin1W, lin1b, lin2W, lin2b):
    G = h.shape[0]
    C = lin2W.shape[1]
    w2 = jnp.zeros((lin2W.shape[0], 128), jnp.float32).at[:, :C].set(lin2W)
    b2 = jnp.full((1, 128), -1e30, jnp.float32).at[0, :C].set(lin2b)
    out = pl.pallas_call(
        _mlp_body,
        out_shape=jax.ShapeDtypeStruct((G, 128), jnp.float32),
    )(h, bn_gamma[None, :], bn_beta[None, :], lin1W, lin1b[None, :], w2, b2)
    return out[:, :C]


def kernel(x, edge_index, batch,
           blk0_W0, blk0_b0, blk0_W1, blk0_b1, blk0_linW, blk0_linb,
           blk1_W0, blk1_b0, blk1_W1, blk1_b1, blk1_linW, blk1_linb,
           blk2_W0, blk2_b0, blk2_W1, blk2_b1, blk2_linW, blk2_linb,
           pool1_p, pool2_p, bn_gamma, bn_beta, lin1W, lin1b, lin2W, lin2b):
    P = dict(
        blk0_W0=blk0_W0, blk0_b0=blk0_b0, blk0_W1=blk0_W1, blk0_b1=blk0_b1,
        blk0_linW=blk0_linW, blk0_linb=blk0_linb,
        blk1_W0=blk1_W0, blk1_b0=blk1_b0, blk1_W1=blk1_W1, blk1_b1=blk1_b1,
        blk1_linW=blk1_linW, blk1_linb=blk1_linb,
        blk2_W0=blk2_W0, blk2_b0=blk2_b0, blk2_W1=blk2_W1, blk2_b1=blk2_b1,
        blk2_linW=blk2_linW, blk2_linb=blk2_linb,
        pool1_p=pool1_p, pool2_p=pool2_p,
    )
    G = N_GRAPHS
    n = x.shape[0]
    nmask = jnp.ones((n,), bool)
    emask = jnp.ones((edge_index.shape[1],), bool)
    x = _block(x, edge_index[0], edge_index[1], emask,
               P["blk0_W0"], P["blk0_b0"], P["blk0_W1"], P["blk0_b1"],
               P["blk0_linW"], P["blk0_linb"])
    xs = [_gadd(x, batch, nmask, G), _gmax(x, batch, nmask, G)]
    ei = edge_index
    for b in (1, 2):
        x, ei, batch, nmask, emask = _topk_pool(
            x, ei, emask, batch, nmask, P["pool%d_p" % b], RATIO, G)
        x = _block(x, ei[0], ei[1], emask,
                   P["blk%d_W0" % b], P["blk%d_b0" % b],
                   P["blk%d_W1" % b], P["blk%d_b1" % b],
                   P["blk%d_linW" % b], P["blk%d_linb" % b])
        xs.extend([_gadd(x, batch, nmask, G), _gmax(x, batch, nmask, G)])
    h = jnp.concatenate(xs, axis=1)
    return _mlp_head(h, bn_gamma, bn_beta, lin1W, lin1b, lin2W, lin2b)


# speedup vs baseline: 1.6220x; 1.6220x over previous
import functools
import jax
import jax.numpy as jnp
from jax import lax
from jax.experimental import pallas as pl
from jax.experimental.pallas import tpu as pltpu
from jax.experimental.pallas import tpu_sc as plsc

N_GRAPHS = 64
RATIO = 0.8
EPS = 1e-5

_NC, _NS = 2, 16
_NP = 10240
_TRASH = 10000
_E = 320000
_EPT = _E // (_NC * _NS)
_CE = 128
_NFULL = _EPT // _CE
_CT = _EPT - _NFULL * _CE


def _agg_sc_body(h_hbm, src_hbm, dst_hbm, z_hbm, out_hbm,
                 is0, id0, r0v, is1, id1, r1v, ist, idt, rtv,
                 acc_sh, gs0, gs1):
    c = lax.axis_index("c")
    s = lax.axis_index("s")
    rpt = _NP // _NS
    row0 = s * rpt
    pltpu.sync_copy(z_hbm.at[pl.ds(row0, rpt)], acc_sh.at[pl.ds(row0, rpt)])
    plsc.subcore_barrier()
    base = (c * _NS + s) * _EPT

    def issue(off, is_v, id_v, rows_v, sem):
        pltpu.sync_copy(src_hbm.at[pl.ds(off, _CE)], is_v)
        pltpu.sync_copy(dst_hbm.at[pl.ds(off, _CE)], id_v)
        pltpu.async_copy(h_hbm.at[is_v], rows_v, sem)

    def drain_scatter(is_v, id_v, rows_v, sem):
        pltpu.make_async_copy(h_hbm.at[is_v], rows_v, sem).wait()
        pltpu.sync_copy(rows_v, acc_sh.at[id_v], add=True)

    issue(base, is0, id0, r0v, gs0)

    def pair(j, carry):
        a = base + (2 * j) * _CE
        issue(a + _CE, is1, id1, r1v, gs1)
        drain_scatter(is0, id0, r0v, gs0)

        @pl.when(j < (_NFULL // 2 - 1))
        def _():
            issue(a + 2 * _CE, is0, id0, r0v, gs0)

        drain_scatter(is1, id1, r1v, gs1)
        return carry

    lax.fori_loop(0, _NFULL // 2, pair, 0)
    offt = base + _NFULL * _CE
    pltpu.sync_copy(src_hbm.at[pl.ds(offt, _CT)], ist)
    pltpu.sync_copy(dst_hbm.at[pl.ds(offt, _CT)], idt)
    pltpu.async_copy(h_hbm.at[ist], rtv, gs0).wait()
    pltpu.sync_copy(rtv, acc_sh.at[idt], add=True)
    plsc.subcore_barrier()
    pltpu.sync_copy(acc_sh.at[pl.ds(row0, rpt)],
                    out_hbm.at[c].at[pl.ds(row0, rpt)])


def _make_agg():
    mesh = plsc.VectorSubcoreMesh(core_axis_name="c", subcore_axis_name="s")
    return functools.partial(
        pl.kernel, mesh=mesh,
        out_type=jax.ShapeDtypeStruct((_NC, _NP, 128), jnp.float32),
        scratch_types=[
            pltpu.VMEM((_CE,), jnp.int32),
            pltpu.VMEM((_CE,), jnp.int32),
            pltpu.VMEM((_CE, 128), jnp.float32),
            pltpu.VMEM((_CE,), jnp.int32),
            pltpu.VMEM((_CE,), jnp.int32),
            pltpu.VMEM((_CE, 128), jnp.float32),
            pltpu.VMEM((_CT,), jnp.int32),
            pltpu.VMEM((_CT,), jnp.int32),
            pltpu.VMEM((_CT, 128), jnp.float32),
            pltpu.VMEM_SHARED((_NP, 128), jnp.float32),
            pltpu.SemaphoreType.DMA,
            pltpu.SemaphoreType.DMA,
        ])(_agg_sc_body)


def _agg_rows(table, src, dst):
    zeros = jnp.zeros((_NP, 128), jnp.float32)
    return _make_agg()(table, src, dst, zeros)


_DW = 128


def _deg_sc_body(ones_hbm, dstm_hbm, z_hbm, out_hbm,
                 id0, id1, idt, ones_v, onest_v, acc_sh, gs0, gs1):
    c = lax.axis_index("c")
    s = lax.axis_index("s")
    rpt = _NP // _NS
    row0 = s * rpt
    pltpu.sync_copy(z_hbm.at[pl.ds(row0, rpt)], acc_sh.at[pl.ds(row0, rpt)])
    pltpu.sync_copy(ones_hbm.at[pl.ds(0, _CE)], ones_v)
    pltpu.sync_copy(ones_hbm.at[pl.ds(0, _CT)], onest_v)
    plsc.subcore_barrier()
    base = (c * _NS + s) * _EPT
    pltpu.async_copy(dstm_hbm.at[pl.ds(base, _CE)], id0, gs0)

    def pair(j, carry):
        a = base + (2 * j) * _CE
        pltpu.async_copy(dstm_hbm.at[pl.ds(a + _CE, _CE)], id1, gs1)
        pltpu.make_async_copy(dstm_hbm.at[pl.ds(a, _CE)], id0, gs0).wait()
        pltpu.sync_copy(ones_v, acc_sh.at[id0], add=True)

        @pl.when(j < (_NFULL // 2 - 1))
        def _():
            pltpu.async_copy(dstm_hbm.at[pl.ds(a + 2 * _CE, _CE)], id0, gs0)

        pltpu.make_async_copy(dstm_hbm.at[pl.ds(a + _CE, _CE)], id1, gs1).wait()
        pltpu.sync_copy(ones_v, acc_sh.at[id1], add=True)
        return carry

    lax.fori_loop(0, _NFULL // 2, pair, 0)
    offt = base + _NFULL * _CE
    pltpu.sync_copy(dstm_hbm.at[pl.ds(offt, _CT)], idt)
    pltpu.sync_copy(onest_v, acc_sh.at[idt], add=True)
    plsc.subcore_barrier()
    pltpu.sync_copy(acc_sh.at[pl.ds(row0, rpt)],
                    out_hbm.at[c].at[pl.ds(row0, rpt)])


def _make_deg():
    mesh = plsc.VectorSubcoreMesh(core_axis_name="c", subcore_axis_name="s")
    return functools.partial(
        pl.kernel, mesh=mesh,
        out_type=jax.ShapeDtypeStruct((_NC, _NP, _DW), jnp.float32),
        scratch_types=[
            pltpu.VMEM((_CE,), jnp.int32),
            pltpu.VMEM((_CE,), jnp.int32),
            pltpu.VMEM((_CT,), jnp.int32),
            pltpu.VMEM((_CE, _DW), jnp.float32),
            pltpu.VMEM((_CT, _DW), jnp.float32),
            pltpu.VMEM_SHARED((_NP, _DW), jnp.float32),
            pltpu.SemaphoreType.DMA,
            pltpu.SemaphoreType.DMA,
        ])(_deg_sc_body)


def _deg_counts(dstm):
    ones = jnp.ones((_CE, _DW), jnp.float32)
    zeros = jnp.zeros((_NP, _DW), jnp.float32)
    return _make_deg()(ones, dstm, zeros)


def _mm_scale_body(x_ref, w_ref, dis_ref, o_ref):
    o_ref[...] = jnp.dot(x_ref[...], w_ref[...],
                         preferred_element_type=jnp.float32) * dis_ref[...]


def _mm_scale(x, W, dis_col):
    BR = 1024
    return pl.pallas_call(
        _mm_scale_body,
        grid=(_NP // BR,),
        in_specs=[pl.BlockSpec((BR, 128), lambda i: (i, 0)),
                  pl.BlockSpec((128, 128), lambda i: (0, 0)),
                  pl.BlockSpec((BR, 1), lambda i: (i, 0))],
        out_specs=pl.BlockSpec((BR, 128), lambda i: (i, 0)),
        out_shape=jax.ShapeDtypeStruct((_NP, 128), jnp.float32),
    )(x, W, dis_col)


def _layer_mid_body(a0_ref, a1_ref, hp_ref, dis_ref, b_ref, w_ref,
                    x1_ref, h2_ref):
    agg = a0_ref[...] + a1_ref[...] + hp_ref[...]
    x1 = jnp.maximum(dis_ref[...] * agg + b_ref[...], 0.0)
    x1_ref[...] = x1
    h2_ref[...] = jnp.dot(x1, w_ref[...],
                          preferred_element_type=jnp.float32) * dis_ref[...]


def _layer_mid(a0, a1, hp, dis_col, b, W):
    BR = 1024
    return pl.pallas_call(
        _layer_mid_body,
        grid=(_NP // BR,),
        in_specs=[pl.BlockSpec((BR, 128), lambda i: (i, 0)),
                  pl.BlockSpec((BR, 128), lambda i: (i, 0)),
                  pl.BlockSpec((BR, 128), lambda i: (i, 0)),
                  pl.BlockSpec((BR, 1), lambda i: (i, 0)),
                  pl.BlockSpec((1, 128), lambda i: (0, 0)),
                  pl.BlockSpec((128, 128), lambda i: (0, 0))],
        out_specs=[pl.BlockSpec((BR, 128), lambda i: (i, 0)),
                   pl.BlockSpec((BR, 128), lambda i: (i, 0))],
        out_shape=[jax.ShapeDtypeStruct((_NP, 128), jnp.float32),
                   jax.ShapeDtypeStruct((_NP, 128), jnp.float32)],
    )(a0, a1, hp, dis_col, b, W)


def _blk_out_body(a0_ref, a1_ref, hp_ref, dis_ref, b_ref, x1_ref,
                  w1_ref, w2_ref, lb_ref, o_ref):
    x2 = jnp.maximum(
        dis_ref[...] * (a0_ref[...] + a1_ref[...] + hp_ref[...]) + b_ref[...],
        0.0)
    o_ref[...] = jnp.maximum(
        jnp.dot(x1_ref[...], w1_ref[...], preferred_element_type=jnp.float32)
        + jnp.dot(x2, w2_ref[...], preferred_element_type=jnp.float32)
        + lb_ref[...], 0.0)


def _blk_out(a0, a1, hp, dis_col, b, x1, linW, linb):
    BR = 1024
    return pl.pallas_call(
        _blk_out_body,
        grid=(_NP // BR,),
        in_specs=[pl.BlockSpec((BR, 128), lambda i: (i, 0)),
                  pl.BlockSpec((BR, 128), lambda i: (i, 0)),
                  pl.BlockSpec((BR, 128), lambda i: (i, 0)),
                  pl.BlockSpec((BR, 1), lambda i: (i, 0)),
                  pl.BlockSpec((1, 128), lambda i: (0, 0)),
                  pl.BlockSpec((BR, 128), lambda i: (i, 0)),
                  pl.BlockSpec((128, 128), lambda i: (0, 0)),
                  pl.BlockSpec((128, 128), lambda i: (0, 0)),
                  pl.BlockSpec((1, 128), lambda i: (0, 0))],
        out_specs=pl.BlockSpec((BR, 128), lambda i: (i, 0)),
        out_shape=jax.ShapeDtypeStruct((_NP, 128), jnp.float32),
    )(a0, a1, hp, dis_col, b, x1, linW[:128], linW[128:], linb)


def _block(x, src, dst, emask, W0, b0, W1, b1, linW, linb):
    n = x.shape[0]
    dstm = jnp.where(emask, dst, _TRASH).astype(jnp.int32)
    dparts = _deg_counts(dstm)
    deg = 1.0 + (dparts[0, :n, 0] + dparts[1, :n, 0])
    dis = deg ** -0.5
    dis_col = jnp.zeros((_NP, 1), jnp.float32).at[:n, 0].set(dis)

    xpad = jnp.zeros((_NP, 128), jnp.float32).at[:n].set(x)
    h1 = _mm_scale(xpad, W0, dis_col)
    a = _agg_rows(h1, src, dstm)
    x1, h2 = _layer_mid(a[0], a[1], h1, dis_col, b0.reshape(1, 128), W1)
    a2 = _agg_rows(h2, src, dstm)
    out = _blk_out(a2[0], a2[1], h2, dis_col, b1.reshape(1, 128), x1,
                   linW, linb.reshape(1, 128))
    return out[:n]


def _counts_body(b_ref, v_ref, ratio_ref, k_ref):
    Np = b_ref.shape[1]
    G = k_ref.shape[0]
    gi = lax.broadcasted_iota(jnp.int32, (G, 1), 0).astype(jnp.float32)
    acc = jnp.zeros((G, 1), jnp.float32)
    BJ = 2048
    for c in range(Np // BJ):
        bj = b_ref[:, pl.ds(c * BJ, BJ)]
        vj = v_ref[:, pl.ds(c * BJ, BJ)]
        oh = jnp.where(bj == gi, vj, 0.0)
        acc = acc + jnp.sum(oh, axis=1, keepdims=True)
    k_ref[...] = jnp.ceil(ratio_ref[0, 0] * acc)


def _rank_body(sc_ref, bc_ref, vc_ref, sr_ref, br_ref, vr_ref,
               k_ref, koff_ref, tgt_ref, kept_ref, *, n_sentinel):
    BI = sc_ref.shape[0]
    Np = sr_ref.shape[1]
    G = k_ref.shape[1]
    i0 = pl.program_id(0) * BI
    si = sc_ref[...]
    bi = bc_ref[...]
    vi = vc_ref[...]
    ii = i0 + lax.broadcasted_iota(jnp.int32, (BI, 1), 0).astype(jnp.float32)
    acc = jnp.zeros((BI, 1), jnp.float32)
    BJ = 2048
    for c in range(Np // BJ):
        sj = sr_ref[:, pl.ds(c * BJ, BJ)]
        bj = br_ref[:, pl.ds(c * BJ, BJ)]
        vj = vr_ref[:, pl.ds(c * BJ, BJ)]
        jj = c * BJ + lax.broadcasted_iota(jnp.int32, (1, BJ), 1).astype(jnp.float32)
        beats = (sj > si) | ((sj == si) & (jj < ii))
        cmp = (bj == bi) & (vj > 0.0) & beats
        acc = acc + jnp.sum(cmp.astype(jnp.float32), axis=1, keepdims=True)
    ohm = bi == lax.broadcasted_iota(jnp.int32, (1, G), 1).astype(jnp.float32)
    kb = jnp.sum(jnp.where(ohm, k_ref[...], 0.0), axis=1, keepdims=True)
    kob = jnp.sum(jnp.where(ohm, koff_ref[...], 0.0), axis=1, keepdims=True)
    kept = (vi > 0.0) & (acc < kb)
    tgt_ref[...] = jnp.where(kept, kob + acc, float(n_sentinel))
    kept_ref[...] = kept.astype(jnp.float32)


def _topk_pool(x, edge_index, edge_mask, batch, node_mask, p, ratio, num_graphs):
    n, D = x.shape
    G = num_graphs
    Np = ((n + 2047) // 2048) * 2048
    bpad = jnp.zeros((Np, 1), jnp.float32).at[:n, 0].set(batch.astype(jnp.float32))
    vpad = jnp.zeros((Np, 1), jnp.float32).at[:n, 0].set(node_mask.astype(jnp.float32))

    scr = jnp.tanh((x @ p) / jnp.linalg.norm(p))
    scr_col = jnp.zeros((Np, 1), jnp.float32).at[:n, 0].set(scr)

    b_row = bpad.reshape(1, Np)
    v_row = vpad.reshape(1, Np)
    s_row = scr_col.reshape(1, Np)
    ratio_arr = jnp.full((1, 1), ratio, jnp.float32)

    k_col = pl.pallas_call(
        _counts_body,
        out_shape=jax.ShapeDtypeStruct((G, 1), jnp.float32),
    )(b_row, v_row, ratio_arr)
    k_row = k_col.reshape(1, G)
    koff_full = jnp.concatenate([jnp.zeros((1,), jnp.float32),
                                 jnp.cumsum(k_col[:, 0])])
    koff_row = koff_full[:G].reshape(1, G)
    total_kept = koff_full[G].astype(jnp.int32)

    BI = 256
    tgt_col, kept_col = pl.pallas_call(
        functools.partial(_rank_body, n_sentinel=n),
        grid=(Np // BI,),
        in_specs=[pl.BlockSpec((BI, 1), lambda i: (i, 0)),
                  pl.BlockSpec((BI, 1), lambda i: (i, 0)),
                  pl.BlockSpec((BI, 1), lambda i: (i, 0)),
                  pl.BlockSpec((1, Np), lambda i: (0, 0)),
                  pl.BlockSpec((1, Np), lambda i: (0, 0)),
                  pl.BlockSpec((1, Np), lambda i: (0, 0)),
                  pl.BlockSpec((1, G), lambda i: (0, 0)),
                  pl.BlockSpec((1, G), lambda i: (0, 0))],
        out_specs=[pl.BlockSpec((BI, 1), lambda i: (i, 0)),
                   pl.BlockSpec((BI, 1), lambda i: (i, 0))],
        out_shape=[jax.ShapeDtypeStruct((Np, 1), jnp.float32),
                   jax.ShapeDtypeStruct((Np, 1), jnp.float32)],
    )(scr_col, bpad, vpad, s_row, b_row, v_row, k_row, koff_row)

    tgt = tgt_col[:n, 0].astype(jnp.int32)
    kept = kept_col[:n, 0] > 0.0

    xn = jnp.zeros((n + 1, D), x.dtype).at[tgt].set(x * scr[:, None])[:n]
    bnew = jnp.zeros((n + 1,), batch.dtype).at[tgt].set(batch)[:n]
    nmask_new = jnp.arange(n, dtype=jnp.int32) < total_kept
    s, d = edge_index[0], edge_index[1]
    em = edge_mask & kept[s] & kept[d]
    src_new = jnp.where(em, tgt[s], 0).astype(s.dtype)
    dst_new = jnp.where(em, tgt[d], 0).astype(d.dtype)
    einew = jnp.stack([src_new, dst_new])
    return xn, einew, bnew, nmask_new, em


def _gadd(x, b, nmask, G):
    seg = jnp.where(nmask, b, G)
    return jnp.zeros((G + 1, x.shape[1]), x.dtype).at[seg].add(x)[:G]


def _gmax(x, b, nmask, G):
    seg = jnp.where(nmask, b, G)
    return jax.ops.segment_max(x, seg, num_segments=G + 1)[:G]


def _mlp_body(h_ref, g_ref, be_ref, w1_ref, b1_ref, w2_ref, b2_ref, o_ref):
    h = h_ref[...]
    h = (h / jnp.sqrt(1.0 + EPS)) * g_ref[...] + be_ref[...]
    h = jnp.maximum(jnp.dot(h, w1_ref[...], preferred_element_type=jnp.float32)
                    + b1_ref[...], 0.0)
    z = jnp.dot(h, w2_ref[...], preferred_element_type=jnp.float32) + b2_ref[...]
    z = z - jnp.max(z, axis=-1, keepdims=True)
    e = jnp.exp(z)
    o_ref[...] = e / jnp.sum(e, axis=-1, keepdims=True)


def _mlp_head(h, bn_gamma, bn_beta, lin1W, lin1b, lin2W, lin2b):
    G = h.shape[0]
    C = lin2W.shape[1]
    w2 = jnp.zeros((lin2W.shape[0], 128), jnp.float32).at[:, :C].set(lin2W)
    b2 = jnp.full((1, 128), -1e30, jnp.float32).at[0, :C].set(lin2b)
    out = pl.pallas_call(
        _mlp_body,
        out_shape=jax.ShapeDtypeStruct((G, 128), jnp.float32),
    )(h, bn_gamma[None, :], bn_beta[None, :], lin1W, lin1b[None, :], w2, b2)
    return out[:, :C]


def kernel(x, edge_index, batch,
           blk0_W0, blk0_b0, blk0_W1, blk0_b1, blk0_linW, blk0_linb,
           blk1_W0, blk1_b0, blk1_W1, blk1_b1, blk1_linW, blk1_linb,
           blk2_W0, blk2_b0, blk2_W1, blk2_b1, blk2_linW, blk2_linb,
           pool1_p, pool2_p, bn_gamma, bn_beta, lin1W, lin1b, lin2W, lin2b):
    P = dict(
        blk0_W0=blk0_W0, blk0_b0=blk0_b0, blk0_W1=blk0_W1, blk0_b1=blk0_b1,
        blk0_linW=blk0_linW, blk0_linb=blk0_linb,
        blk1_W0=blk1_W0, blk1_b0=blk1_b0, blk1_W1=blk1_W1, blk1_b1=blk1_b1,
        blk1_linW=blk1_linW, blk1_linb=blk1_linb,
        blk2_W0=blk2_W0, blk2_b0=blk2_b0, blk2_W1=blk2_W1, blk2_b1=blk2_b1,
        blk2_linW=blk2_linW, blk2_linb=blk2_linb,
        pool1_p=pool1_p, pool2_p=pool2_p,
    )
    G = N_GRAPHS
    n = x.shape[0]
    nmask = jnp.ones((n,), bool)
    emask = jnp.ones((edge_index.shape[1],), bool)
    x = _block(x, edge_index[0], edge_index[1], emask,
               P["blk0_W0"], P["blk0_b0"], P["blk0_W1"], P["blk0_b1"],
               P["blk0_linW"], P["blk0_linb"])
    xs = [_gadd(x, batch, nmask, G), _gmax(x, batch, nmask, G)]
    ei = edge_index
    for b in (1, 2):
        x, ei, batch, nmask, emask = _topk_pool(
            x, ei, emask, batch, nmask, P["pool%d_p" % b], RATIO, G)
        x = _block(x, ei[0], ei[1], emask,
                   P["blk%d_W0" % b], P["blk%d_b0" % b],
                   P["blk%d_W1" % b], P["blk%d_b1" % b],
                   P["blk%d_linW" % b], P["blk%d_linb" % b])
        xs.extend([_gadd(x, batch, nmask, G), _gmax(x, batch, nmask, G)])
    h = jnp.concatenate(xs, axis=1)
    return _mlp_head(h, bn_gamma, bn_beta, lin1W, lin1b, lin2W, lin2b)


# speedup vs baseline: 1.6270x; 1.0031x over previous
import functools
import jax
import jax.numpy as jnp
from jax import lax
from jax.experimental import pallas as pl
from jax.experimental.pallas import tpu as pltpu
from jax.experimental.pallas import tpu_sc as plsc

N_GRAPHS = 64
RATIO = 0.8
EPS = 1e-5

_NC, _NS = 2, 16
_NP = 10240
_TRASH = 10000
_E = 320000
_EPT = _E // (_NC * _NS)
_CE = 128
_NFULL = _EPT // _CE
_CT = _EPT - _NFULL * _CE


def _agg_sc_body(h_hbm, src_hbm, dst_hbm, z_hbm, out_hbm,
                 is0, id0, r0v, is1, id1, r1v, ist, idt, rtv,
                 acc_sh, gs0, gs1):
    c = lax.axis_index("c")
    s = lax.axis_index("s")
    rpt = _NP // _NS
    row0 = s * rpt
    pltpu.sync_copy(z_hbm.at[pl.ds(row0, rpt)], acc_sh.at[pl.ds(row0, rpt)])
    plsc.subcore_barrier()
    base = (c * _NS + s) * _EPT

    def issue(off, is_v, id_v, rows_v, sem):
        pltpu.sync_copy(src_hbm.at[pl.ds(off, _CE)], is_v)
        pltpu.sync_copy(dst_hbm.at[pl.ds(off, _CE)], id_v)
        pltpu.async_copy(h_hbm.at[is_v], rows_v, sem)

    def drain_scatter(is_v, id_v, rows_v, sem):
        pltpu.make_async_copy(h_hbm.at[is_v], rows_v, sem).wait()
        pltpu.sync_copy(rows_v, acc_sh.at[id_v], add=True)

    issue(base, is0, id0, r0v, gs0)

    def pair(j, carry):
        a = base + (2 * j) * _CE
        issue(a + _CE, is1, id1, r1v, gs1)
        drain_scatter(is0, id0, r0v, gs0)

        @pl.when(j < (_NFULL // 2 - 1))
        def _():
            issue(a + 2 * _CE, is0, id0, r0v, gs0)

        drain_scatter(is1, id1, r1v, gs1)
        return carry

    lax.fori_loop(0, _NFULL // 2, pair, 0)
    offt = base + _NFULL * _CE
    pltpu.sync_copy(src_hbm.at[pl.ds(offt, _CT)], ist)
    pltpu.sync_copy(dst_hbm.at[pl.ds(offt, _CT)], idt)
    pltpu.async_copy(h_hbm.at[ist], rtv, gs0).wait()
    pltpu.sync_copy(rtv, acc_sh.at[idt], add=True)
    plsc.subcore_barrier()
    pltpu.sync_copy(acc_sh.at[pl.ds(row0, rpt)],
                    out_hbm.at[c].at[pl.ds(row0, rpt)])


def _make_agg():
    mesh = plsc.VectorSubcoreMesh(core_axis_name="c", subcore_axis_name="s")
    return functools.partial(
        pl.kernel, mesh=mesh,
        out_type=jax.ShapeDtypeStruct((_NC, _NP, 128), jnp.float32),
        scratch_types=[
            pltpu.VMEM((_CE,), jnp.int32),
            pltpu.VMEM((_CE,), jnp.int32),
            pltpu.VMEM((_CE, 128), jnp.float32),
            pltpu.VMEM((_CE,), jnp.int32),
            pltpu.VMEM((_CE,), jnp.int32),
            pltpu.VMEM((_CE, 128), jnp.float32),
            pltpu.VMEM((_CT,), jnp.int32),
            pltpu.VMEM((_CT,), jnp.int32),
            pltpu.VMEM((_CT, 128), jnp.float32),
            pltpu.VMEM_SHARED((_NP, 128), jnp.float32),
            pltpu.SemaphoreType.DMA,
            pltpu.SemaphoreType.DMA,
        ])(_agg_sc_body)


def _agg_rows(table, src, dst):
    zeros = jnp.zeros((_NP, 128), jnp.float32)
    return _make_agg()(table, src, dst, zeros)


def _deg_sc_body(ones_hbm, dstm_hbm, z_hbm, out_hbm,
                 id0, id1, idt, ones_v, onest_v, acc_sh, gs0, gs1):
    c = lax.axis_index("c")
    s = lax.axis_index("s")
    rpt = _NP // _NS
    row0 = s * rpt
    pltpu.sync_copy(z_hbm.at[pl.ds(row0, rpt)], acc_sh.at[pl.ds(row0, rpt)])
    pltpu.sync_copy(ones_hbm.at[pl.ds(0, _CE)], ones_v)
    pltpu.sync_copy(ones_hbm.at[pl.ds(0, _CT)], onest_v)
    plsc.subcore_barrier()
    base = (c * _NS + s) * _EPT
    pltpu.async_copy(dstm_hbm.at[pl.ds(base, _CE)], id0, gs0)

    def pair(j, carry):
        a = base + (2 * j) * _CE
        pltpu.async_copy(dstm_hbm.at[pl.ds(a + _CE, _CE)], id1, gs1)
        pltpu.make_async_copy(dstm_hbm.at[pl.ds(a, _CE)], id0, gs0).wait()
        pltpu.sync_copy(ones_v, acc_sh.at[id0], add=True)

        @pl.when(j < (_NFULL // 2 - 1))
        def _():
            pltpu.async_copy(dstm_hbm.at[pl.ds(a + 2 * _CE, _CE)], id0, gs0)

        pltpu.make_async_copy(dstm_hbm.at[pl.ds(a + _CE, _CE)], id1, gs1).wait()
        pltpu.sync_copy(ones_v, acc_sh.at[id1], add=True)
        return carry

    lax.fori_loop(0, _NFULL // 2, pair, 0)
    offt = base + _NFULL * _CE
    pltpu.sync_copy(dstm_hbm.at[pl.ds(offt, _CT)], idt)
    pltpu.sync_copy(onest_v, acc_sh.at[idt], add=True)
    plsc.subcore_barrier()
    pltpu.sync_copy(acc_sh.at[pl.ds(row0, rpt)],
                    out_hbm.at[c].at[pl.ds(row0, rpt)])


def _make_deg():
    mesh = plsc.VectorSubcoreMesh(core_axis_name="c", subcore_axis_name="s")
    return functools.partial(
        pl.kernel, mesh=mesh,
        out_type=jax.ShapeDtypeStruct((_NC, _NP), jnp.float32),
        scratch_types=[
            pltpu.VMEM((_CE,), jnp.int32),
            pltpu.VMEM((_CE,), jnp.int32),
            pltpu.VMEM((_CT,), jnp.int32),
            pltpu.VMEM((_CE,), jnp.float32),
            pltpu.VMEM((_CT,), jnp.float32),
            pltpu.VMEM_SHARED((_NP,), jnp.float32),
            pltpu.SemaphoreType.DMA,
            pltpu.SemaphoreType.DMA,
        ])(_deg_sc_body)


def _deg_counts(dstm):
    ones = jnp.ones((_CE,), jnp.float32)
    zeros = jnp.zeros((_NP,), jnp.float32)
    return _make_deg()(ones, dstm, zeros)


def _mm_scale_body(x_ref, w_ref, dis_ref, o_ref):
    o_ref[...] = jnp.dot(x_ref[...], w_ref[...],
                         preferred_element_type=jnp.float32) * dis_ref[...]


def _mm_scale(x, W, dis_col):
    BR = 1024
    return pl.pallas_call(
        _mm_scale_body,
        grid=(_NP // BR,),
        in_specs=[pl.BlockSpec((BR, 128), lambda i: (i, 0)),
                  pl.BlockSpec((128, 128), lambda i: (0, 0)),
                  pl.BlockSpec((BR, 1), lambda i: (i, 0))],
        out_specs=pl.BlockSpec((BR, 128), lambda i: (i, 0)),
        out_shape=jax.ShapeDtypeStruct((_NP, 128), jnp.float32),
    )(x, W, dis_col)


def _layer_mid_body(a0_ref, a1_ref, hp_ref, dis_ref, b_ref, w_ref,
                    x1_ref, h2_ref):
    agg = a0_ref[...] + a1_ref[...] + hp_ref[...]
    x1 = jnp.maximum(dis_ref[...] * agg + b_ref[...], 0.0)
    x1_ref[...] = x1
    h2_ref[...] = jnp.dot(x1, w_ref[...],
                          preferred_element_type=jnp.float32) * dis_ref[...]


def _layer_mid(a0, a1, hp, dis_col, b, W):
    BR = 1024
    return pl.pallas_call(
        _layer_mid_body,
        grid=(_NP // BR,),
        in_specs=[pl.BlockSpec((BR, 128), lambda i: (i, 0)),
                  pl.BlockSpec((BR, 128), lambda i: (i, 0)),
                  pl.BlockSpec((BR, 128), lambda i: (i, 0)),
                  pl.BlockSpec((BR, 1), lambda i: (i, 0)),
                  pl.BlockSpec((1, 128), lambda i: (0, 0)),
                  pl.BlockSpec((128, 128), lambda i: (0, 0))],
        out_specs=[pl.BlockSpec((BR, 128), lambda i: (i, 0)),
                   pl.BlockSpec((BR, 128), lambda i: (i, 0))],
        out_shape=[jax.ShapeDtypeStruct((_NP, 128), jnp.float32),
                   jax.ShapeDtypeStruct((_NP, 128), jnp.float32)],
    )(a0, a1, hp, dis_col, b, W)


def _blk_out_body(a0_ref, a1_ref, hp_ref, dis_ref, b_ref, x1_ref,
                  w1_ref, w2_ref, lb_ref, o_ref):
    x2 = jnp.maximum(
        dis_ref[...] * (a0_ref[...] + a1_ref[...] + hp_ref[...]) + b_ref[...],
        0.0)
    o_ref[...] = jnp.maximum(
        jnp.dot(x1_ref[...], w1_ref[...], preferred_element_type=jnp.float32)
        + jnp.dot(x2, w2_ref[...], preferred_element_type=jnp.float32)
        + lb_ref[...], 0.0)


def _blk_out(a0, a1, hp, dis_col, b, x1, linW, linb):
    BR = 1024
    return pl.pallas_call(
        _blk_out_body,
        grid=(_NP // BR,),
        in_specs=[pl.BlockSpec((BR, 128), lambda i: (i, 0)),
                  pl.BlockSpec((BR, 128), lambda i: (i, 0)),
                  pl.BlockSpec((BR, 128), lambda i: (i, 0)),
                  pl.BlockSpec((BR, 1), lambda i: (i, 0)),
                  pl.BlockSpec((1, 128), lambda i: (0, 0)),
                  pl.BlockSpec((BR, 128), lambda i: (i, 0)),
                  pl.BlockSpec((128, 128), lambda i: (0, 0)),
                  pl.BlockSpec((128, 128), lambda i: (0, 0)),
                  pl.BlockSpec((1, 128), lambda i: (0, 0))],
        out_specs=pl.BlockSpec((BR, 128), lambda i: (i, 0)),
        out_shape=jax.ShapeDtypeStruct((_NP, 128), jnp.float32),
    )(a0, a1, hp, dis_col, b, x1, linW[:128], linW[128:], linb)


def _block(x, src, dst, emask, W0, b0, W1, b1, linW, linb):
    n = x.shape[0]
    dstm = jnp.where(emask, dst, _TRASH).astype(jnp.int32)
    dparts = _deg_counts(dstm)
    deg = 1.0 + (dparts[0, :n] + dparts[1, :n])
    dis = deg ** -0.5
    dis_col = jnp.zeros((_NP, 1), jnp.float32).at[:n, 0].set(dis)

    xpad = jnp.zeros((_NP, 128), jnp.float32).at[:n].set(x)
    h1 = _mm_scale(xpad, W0, dis_col)
    a = _agg_rows(h1, src, dstm)
    x1, h2 = _layer_mid(a[0], a[1], h1, dis_col, b0.reshape(1, 128), W1)
    a2 = _agg_rows(h2, src, dstm)
    out = _blk_out(a2[0], a2[1], h2, dis_col, b1.reshape(1, 128), x1,
                   linW, linb.reshape(1, 128))
    return out[:n]


def _counts_body(b_ref, v_ref, ratio_ref, k_ref):
    Np = b_ref.shape[1]
    G = k_ref.shape[0]
    gi = lax.broadcasted_iota(jnp.int32, (G, 1), 0).astype(jnp.float32)
    acc = jnp.zeros((G, 1), jnp.float32)
    BJ = 2048
    for c in range(Np // BJ):
        bj = b_ref[:, pl.ds(c * BJ, BJ)]
        vj = v_ref[:, pl.ds(c * BJ, BJ)]
        oh = jnp.where(bj == gi, vj, 0.0)
        acc = acc + jnp.sum(oh, axis=1, keepdims=True)
    k_ref[...] = jnp.ceil(ratio_ref[0, 0] * acc)


def _rank_body(sc_ref, bc_ref, vc_ref, sr_ref, br_ref, vr_ref,
               k_ref, koff_ref, tgt_ref, kept_ref, *, n_sentinel):
    BI = sc_ref.shape[0]
    Np = sr_ref.shape[1]
    G = k_ref.shape[1]
    i0 = pl.program_id(0) * BI
    si = sc_ref[...]
    bi = bc_ref[...]
    vi = vc_ref[...]
    ii = i0 + lax.broadcasted_iota(jnp.int32, (BI, 1), 0).astype(jnp.float32)
    acc = jnp.zeros((BI, 1), jnp.float32)
    BJ = 2048
    for c in range(Np // BJ):
        sj = sr_ref[:, pl.ds(c * BJ, BJ)]
        bj = br_ref[:, pl.ds(c * BJ, BJ)]
        vj = vr_ref[:, pl.ds(c * BJ, BJ)]
        jj = c * BJ + lax.broadcasted_iota(jnp.int32, (1, BJ), 1).astype(jnp.float32)
        beats = (sj > si) | ((sj == si) & (jj < ii))
        cmp = (bj == bi) & (vj > 0.0) & beats
        acc = acc + jnp.sum(cmp.astype(jnp.float32), axis=1, keepdims=True)
    ohm = bi == lax.broadcasted_iota(jnp.int32, (1, G), 1).astype(jnp.float32)
    kb = jnp.sum(jnp.where(ohm, k_ref[...], 0.0), axis=1, keepdims=True)
    kob = jnp.sum(jnp.where(ohm, koff_ref[...], 0.0), axis=1, keepdims=True)
    kept = (vi > 0.0) & (acc < kb)
    tgt_ref[...] = jnp.where(kept, kob + acc, float(n_sentinel))
    kept_ref[...] = kept.astype(jnp.float32)


def _topk_pool(x, edge_index, edge_mask, batch, node_mask, p, ratio, num_graphs):
    n, D = x.shape
    G = num_graphs
    Np = ((n + 2047) // 2048) * 2048
    bpad = jnp.zeros((Np, 1), jnp.float32).at[:n, 0].set(batch.astype(jnp.float32))
    vpad = jnp.zeros((Np, 1), jnp.float32).at[:n, 0].set(node_mask.astype(jnp.float32))

    scr = jnp.tanh((x @ p) / jnp.linalg.norm(p))
    scr_col = jnp.zeros((Np, 1), jnp.float32).at[:n, 0].set(scr)

    b_row = bpad.reshape(1, Np)
    v_row = vpad.reshape(1, Np)
    s_row = scr_col.reshape(1, Np)
    ratio_arr = jnp.full((1, 1), ratio, jnp.float32)

    k_col = pl.pallas_call(
        _counts_body,
        out_shape=jax.ShapeDtypeStruct((G, 1), jnp.float32),
    )(b_row, v_row, ratio_arr)
    k_row = k_col.reshape(1, G)
    koff_full = jnp.concatenate([jnp.zeros((1,), jnp.float32),
                                 jnp.cumsum(k_col[:, 0])])
    koff_row = koff_full[:G].reshape(1, G)
    total_kept = koff_full[G].astype(jnp.int32)

    BI = 256
    tgt_col, kept_col = pl.pallas_call(
        functools.partial(_rank_body, n_sentinel=n),
        grid=(Np // BI,),
        in_specs=[pl.BlockSpec((BI, 1), lambda i: (i, 0)),
                  pl.BlockSpec((BI, 1), lambda i: (i, 0)),
                  pl.BlockSpec((BI, 1), lambda i: (i, 0)),
                  pl.BlockSpec((1, Np), lambda i: (0, 0)),
                  pl.BlockSpec((1, Np), lambda i: (0, 0)),
                  pl.BlockSpec((1, Np), lambda i: (0, 0)),
                  pl.BlockSpec((1, G), lambda i: (0, 0)),
                  pl.BlockSpec((1, G), lambda i: (0, 0))],
        out_specs=[pl.BlockSpec((BI, 1), lambda i: (i, 0)),
                   pl.BlockSpec((BI, 1), lambda i: (i, 0))],
        out_shape=[jax.ShapeDtypeStruct((Np, 1), jnp.float32),
                   jax.ShapeDtypeStruct((Np, 1), jnp.float32)],
    )(scr_col, bpad, vpad, s_row, b_row, v_row, k_row, koff_row)

    tgt = tgt_col[:n, 0].astype(jnp.int32)
    kept = kept_col[:n, 0] > 0.0

    xn = jnp.zeros((n + 1, D), x.dtype).at[tgt].set(x * scr[:, None])[:n]
    bnew = jnp.zeros((n + 1,), batch.dtype).at[tgt].set(batch)[:n]
    nmask_new = jnp.arange(n, dtype=jnp.int32) < total_kept
    s, d = edge_index[0], edge_index[1]
    em = edge_mask & kept[s] & kept[d]
    src_new = jnp.where(em, tgt[s], 0).astype(s.dtype)
    dst_new = jnp.where(em, tgt[d], 0).astype(d.dtype)
    einew = jnp.stack([src_new, dst_new])
    return xn, einew, bnew, nmask_new, em


def _gadd(x, b, nmask, G):
    seg = jnp.where(nmask, b, G)
    return jnp.zeros((G + 1, x.shape[1]), x.dtype).at[seg].add(x)[:G]


def _gmax(x, b, nmask, G):
    seg = jnp.where(nmask, b, G)
    return jax.ops.segment_max(x, seg, num_segments=G + 1)[:G]


def _mlp_body(h_ref, g_ref, be_ref, w1_ref, b1_ref, w2_ref, b2_ref, o_ref):
    h = h_ref[...]
    h = (h / jnp.sqrt(1.0 + EPS)) * g_ref[...] + be_ref[...]
    h = jnp.maximum(jnp.dot(h, w1_ref[...], preferred_element_type=jnp.float32)
                    + b1_ref[...], 0.0)
    z = jnp.dot(h, w2_ref[...], preferred_element_type=jnp.float32) + b2_ref[...]
    z = z - jnp.max(z, axis=-1, keepdims=True)
    e = jnp.exp(z)
    o_ref[...] = e / jnp.sum(e, axis=-1, keepdims=True)


def _mlp_head(h, bn_gamma, bn_beta, lin1W, lin1b, lin2W, lin2b):
    G = h.shape[0]
    C = lin2W.shape[1]
    w2 = jnp.zeros((lin2W.shape[0], 128), jnp.float32).at[:, :C].set(lin2W)
    b2 = jnp.full((1, 128), -1e30, jnp.float32).at[0, :C].set(lin2b)
    out = pl.pallas_call(
        _mlp_body,
        out_shape=jax.ShapeDtypeStruct((G, 128), jnp.float32),
    )(h, bn_gamma[None, :], bn_beta[None, :], lin1W, lin1b[None, :], w2, b2)
    return out[:, :C]


def kernel(x, edge_index, batch,
           blk0_W0, blk0_b0, blk0_W1, blk0_b1, blk0_linW, blk0_linb,
           blk1_W0, blk1_b0, blk1_W1, blk1_b1, blk1_linW, blk1_linb,
           blk2_W0, blk2_b0, blk2_W1, blk2_b1, blk2_linW, blk2_linb,
           pool1_p, pool2_p, bn_gamma, bn_beta, lin1W, lin1b, lin2W, lin2b):
    P = dict(
        blk0_W0=blk0_W0, blk0_b0=blk0_b0, blk0_W1=blk0_W1, blk0_b1=blk0_b1,
        blk0_linW=blk0_linW, blk0_linb=blk0_linb,
        blk1_W0=blk1_W0, blk1_b0=blk1_b0, blk1_W1=blk1_W1, blk1_b1=blk1_b1,
        blk1_linW=blk1_linW, blk1_linb=blk1_linb,
        blk2_W0=blk2_W0, blk2_b0=blk2_b0, blk2_W1=blk2_W1, blk2_b1=blk2_b1,
        blk2_linW=blk2_linW, blk2_linb=blk2_linb,
        pool1_p=pool1_p, pool2_p=pool2_p,
    )
    G = N_GRAPHS
    n = x.shape[0]
    nmask = jnp.ones((n,), bool)
    emask = jnp.ones((edge_index.shape[1],), bool)
    x = _block(x, edge_index[0], edge_index[1], emask,
               P["blk0_W0"], P["blk0_b0"], P["blk0_W1"], P["blk0_b1"],
               P["blk0_linW"], P["blk0_linb"])
    xs = [_gadd(x, batch, nmask, G), _gmax(x, batch, nmask, G)]
    ei = edge_index
    for b in (1, 2):
        x, ei, batch, nmask, emask = _topk_pool(
            x, ei, emask, batch, nmask, P["pool%d_p" % b], RATIO, G)
        x = _block(x, ei[0], ei[1], emask,
                   P["blk%d_W0" % b], P["blk%d_b0" % b],
                   P["blk%d_W1" % b], P["blk%d_b1" % b],
                   P["blk%d_linW" % b], P["blk%d_linb" % b])
        xs.extend([_gadd(x, batch, nmask, G), _gmax(x, batch, nmask, G)])
    h = jnp.concatenate(xs, axis=1)
    return _mlp_head(h, bn_gamma, bn_beta, lin1W, lin1b, lin2W, lin2b)


# speedup vs baseline: 2.0948x; 1.2875x over previous
import functools
import jax
import jax.numpy as jnp
from jax import lax
from jax.experimental import pallas as pl
from jax.experimental.pallas import tpu as pltpu
from jax.experimental.pallas import tpu_sc as plsc

N_GRAPHS = 64
RATIO = 0.8
EPS = 1e-5

_NC, _NS = 2, 16
_NP = 10240
_TRASH = 10000
_E = 320000
_EPT = _E // (_NC * _NS)
_CE = 128
_NFULL = _EPT // _CE
_CT = _EPT - _NFULL * _CE


def _agg_sc_body(h_hbm, src_hbm, dst_hbm, z_hbm, out_hbm,
                 is0, id0, r0v, is1, id1, r1v, ist, idt, rtv,
                 acc_sh, gs0, gs1, ss0, ss1):
    c = lax.axis_index("c")
    s = lax.axis_index("s")
    rpt = _NP // _NS
    row0 = s * rpt
    pltpu.sync_copy(z_hbm.at[pl.ds(row0, rpt)], acc_sh.at[pl.ds(row0, rpt)])
    plsc.subcore_barrier()
    base = (c * _NS + s) * _EPT

    def issue(off, is_v, id_v, rows_v, gsem):
        pltpu.sync_copy(src_hbm.at[pl.ds(off, _CE)], is_v)
        pltpu.sync_copy(dst_hbm.at[pl.ds(off, _CE)], id_v)
        pltpu.async_copy(h_hbm.at[is_v], rows_v, gsem)

    issue(base, is0, id0, r0v, gs0)
    issue(base + _CE, is1, id1, r1v, gs1)

    def pair(j, carry):
        a = base + (2 * j) * _CE
        pltpu.make_async_copy(h_hbm.at[is0], r0v, gs0).wait()
        h0 = pltpu.async_copy(r0v, acc_sh.at[id0], ss0, add=True)
        pltpu.make_async_copy(h_hbm.at[is1], r1v, gs1).wait()
        h1 = pltpu.async_copy(r1v, acc_sh.at[id1], ss1, add=True)
        h0.wait()

        @pl.when(j < _NFULL // 2 - 1)
        def _():
            issue(a + 2 * _CE, is0, id0, r0v, gs0)

        h1.wait()

        @pl.when(j < _NFULL // 2 - 1)
        def _():
            issue(a + 3 * _CE, is1, id1, r1v, gs1)

        return carry

    lax.fori_loop(0, _NFULL // 2, pair, 0)
    offt = base + _NFULL * _CE
    pltpu.sync_copy(src_hbm.at[pl.ds(offt, _CT)], ist)
    pltpu.sync_copy(dst_hbm.at[pl.ds(offt, _CT)], idt)
    pltpu.async_copy(h_hbm.at[ist], rtv, gs0).wait()
    pltpu.sync_copy(rtv, acc_sh.at[idt], add=True)
    plsc.subcore_barrier()
    pltpu.sync_copy(acc_sh.at[pl.ds(row0, rpt)],
                    out_hbm.at[c].at[pl.ds(row0, rpt)])


def _make_agg():
    mesh = plsc.VectorSubcoreMesh(core_axis_name="c", subcore_axis_name="s")
    return functools.partial(
        pl.kernel, mesh=mesh,
        out_type=jax.ShapeDtypeStruct((_NC, _NP, 128), jnp.float32),
        scratch_types=[
            pltpu.VMEM((_CE,), jnp.int32),
            pltpu.VMEM((_CE,), jnp.int32),
            pltpu.VMEM((_CE, 128), jnp.float32),
            pltpu.VMEM((_CE,), jnp.int32),
            pltpu.VMEM((_CE,), jnp.int32),
            pltpu.VMEM((_CE, 128), jnp.float32),
            pltpu.VMEM((_CT,), jnp.int32),
            pltpu.VMEM((_CT,), jnp.int32),
            pltpu.VMEM((_CT, 128), jnp.float32),
            pltpu.VMEM_SHARED((_NP, 128), jnp.float32),
            pltpu.SemaphoreType.DMA,
            pltpu.SemaphoreType.DMA,
            pltpu.SemaphoreType.DMA,
            pltpu.SemaphoreType.DMA,
        ])(_agg_sc_body)


def _agg_rows(table, src, dst):
    zeros = jnp.zeros((_NP, 128), jnp.float32)
    return _make_agg()(table, src, dst, zeros)


def _deg_sc_body(ones_hbm, dstm_hbm, z_hbm, out_hbm,
                 id0, id1, idt, ones_v, onest_v, acc_sh, gs0, gs1):
    c = lax.axis_index("c")
    s = lax.axis_index("s")
    rpt = _NP // _NS
    row0 = s * rpt
    pltpu.sync_copy(z_hbm.at[pl.ds(row0, rpt)], acc_sh.at[pl.ds(row0, rpt)])
    pltpu.sync_copy(ones_hbm.at[pl.ds(0, _CE)], ones_v)
    pltpu.sync_copy(ones_hbm.at[pl.ds(0, _CT)], onest_v)
    plsc.subcore_barrier()
    base = (c * _NS + s) * _EPT
    pltpu.async_copy(dstm_hbm.at[pl.ds(base, _CE)], id0, gs0)

    def pair(j, carry):
        a = base + (2 * j) * _CE
        pltpu.async_copy(dstm_hbm.at[pl.ds(a + _CE, _CE)], id1, gs1)
        pltpu.make_async_copy(dstm_hbm.at[pl.ds(a, _CE)], id0, gs0).wait()
        pltpu.sync_copy(ones_v, acc_sh.at[id0], add=True)

        @pl.when(j < (_NFULL // 2 - 1))
        def _():
            pltpu.async_copy(dstm_hbm.at[pl.ds(a + 2 * _CE, _CE)], id0, gs0)

        pltpu.make_async_copy(dstm_hbm.at[pl.ds(a + _CE, _CE)], id1, gs1).wait()
        pltpu.sync_copy(ones_v, acc_sh.at[id1], add=True)
        return carry

    lax.fori_loop(0, _NFULL // 2, pair, 0)
    offt = base + _NFULL * _CE
    pltpu.sync_copy(dstm_hbm.at[pl.ds(offt, _CT)], idt)
    pltpu.sync_copy(onest_v, acc_sh.at[idt], add=True)
    plsc.subcore_barrier()
    pltpu.sync_copy(acc_sh.at[pl.ds(row0, rpt)],
                    out_hbm.at[c].at[pl.ds(row0, rpt)])


def _make_deg():
    mesh = plsc.VectorSubcoreMesh(core_axis_name="c", subcore_axis_name="s")
    return functools.partial(
        pl.kernel, mesh=mesh,
        out_type=jax.ShapeDtypeStruct((_NC, _NP), jnp.float32),
        scratch_types=[
            pltpu.VMEM((_CE,), jnp.int32),
            pltpu.VMEM((_CE,), jnp.int32),
            pltpu.VMEM((_CT,), jnp.int32),
            pltpu.VMEM((_CE,), jnp.float32),
            pltpu.VMEM((_CT,), jnp.float32),
            pltpu.VMEM_SHARED((_NP,), jnp.float32),
            pltpu.SemaphoreType.DMA,
            pltpu.SemaphoreType.DMA,
        ])(_deg_sc_body)


def _deg_counts(dstm):
    ones = jnp.ones((_CE,), jnp.float32)
    zeros = jnp.zeros((_NP,), jnp.float32)
    return _make_deg()(ones, dstm, zeros)


def _mm_scale_body(x_ref, w_ref, dis_ref, o_ref):
    o_ref[...] = jnp.dot(x_ref[...], w_ref[...],
                         preferred_element_type=jnp.float32) * dis_ref[...]


def _mm_scale(x, W, dis_col):
    BR = 1024
    return pl.pallas_call(
        _mm_scale_body,
        grid=(_NP // BR,),
        in_specs=[pl.BlockSpec((BR, 128), lambda i: (i, 0)),
                  pl.BlockSpec((128, 128), lambda i: (0, 0)),
                  pl.BlockSpec((BR, 1), lambda i: (i, 0))],
        out_specs=pl.BlockSpec((BR, 128), lambda i: (i, 0)),
        out_shape=jax.ShapeDtypeStruct((_NP, 128), jnp.float32),
    )(x, W, dis_col)


def _layer_mid_body(a0_ref, a1_ref, hp_ref, dis_ref, b_ref, w_ref,
                    x1_ref, h2_ref):
    agg = a0_ref[...] + a1_ref[...] + hp_ref[...]
    x1 = jnp.maximum(dis_ref[...] * agg + b_ref[...], 0.0)
    x1_ref[...] = x1
    h2_ref[...] = jnp.dot(x1, w_ref[...],
                          preferred_element_type=jnp.float32) * dis_ref[...]


def _layer_mid(a0, a1, hp, dis_col, b, W):
    BR = 1024
    return pl.pallas_call(
        _layer_mid_body,
        grid=(_NP // BR,),
        in_specs=[pl.BlockSpec((BR, 128), lambda i: (i, 0)),
                  pl.BlockSpec((BR, 128), lambda i: (i, 0)),
                  pl.BlockSpec((BR, 128), lambda i: (i, 0)),
                  pl.BlockSpec((BR, 1), lambda i: (i, 0)),
                  pl.BlockSpec((1, 128), lambda i: (0, 0)),
                  pl.BlockSpec((128, 128), lambda i: (0, 0))],
        out_specs=[pl.BlockSpec((BR, 128), lambda i: (i, 0)),
                   pl.BlockSpec((BR, 128), lambda i: (i, 0))],
        out_shape=[jax.ShapeDtypeStruct((_NP, 128), jnp.float32),
                   jax.ShapeDtypeStruct((_NP, 128), jnp.float32)],
    )(a0, a1, hp, dis_col, b, W)


def _blk_out_body(a0_ref, a1_ref, hp_ref, dis_ref, b_ref, x1_ref,
                  w1_ref, w2_ref, lb_ref, o_ref):
    x2 = jnp.maximum(
        dis_ref[...] * (a0_ref[...] + a1_ref[...] + hp_ref[...]) + b_ref[...],
        0.0)
    o_ref[...] = jnp.maximum(
        jnp.dot(x1_ref[...], w1_ref[...], preferred_element_type=jnp.float32)
        + jnp.dot(x2, w2_ref[...], preferred_element_type=jnp.float32)
        + lb_ref[...], 0.0)


def _blk_out(a0, a1, hp, dis_col, b, x1, linW, linb):
    BR = 1024
    return pl.pallas_call(
        _blk_out_body,
        grid=(_NP // BR,),
        in_specs=[pl.BlockSpec((BR, 128), lambda i: (i, 0)),
                  pl.BlockSpec((BR, 128), lambda i: (i, 0)),
                  pl.BlockSpec((BR, 128), lambda i: (i, 0)),
                  pl.BlockSpec((BR, 1), lambda i: (i, 0)),
                  pl.BlockSpec((1, 128), lambda i: (0, 0)),
                  pl.BlockSpec((BR, 128), lambda i: (i, 0)),
                  pl.BlockSpec((128, 128), lambda i: (0, 0)),
                  pl.BlockSpec((128, 128), lambda i: (0, 0)),
                  pl.BlockSpec((1, 128), lambda i: (0, 0))],
        out_specs=pl.BlockSpec((BR, 128), lambda i: (i, 0)),
        out_shape=jax.ShapeDtypeStruct((_NP, 128), jnp.float32),
    )(a0, a1, hp, dis_col, b, x1, linW[:128], linW[128:], linb)


def _block(x, src, dst, emask, W0, b0, W1, b1, linW, linb):
    n = x.shape[0]
    dstm = jnp.where(emask, dst, _TRASH).astype(jnp.int32)
    dparts = _deg_counts(dstm)
    deg = 1.0 + (dparts[0, :n] + dparts[1, :n])
    dis = deg ** -0.5
    dis_col = jnp.zeros((_NP, 1), jnp.float32).at[:n, 0].set(dis)

    xpad = jnp.zeros((_NP, 128), jnp.float32).at[:n].set(x)
    h1 = _mm_scale(xpad, W0, dis_col)
    a = _agg_rows(h1, src, dstm)
    x1, h2 = _layer_mid(a[0], a[1], h1, dis_col, b0.reshape(1, 128), W1)
    a2 = _agg_rows(h2, src, dstm)
    out = _blk_out(a2[0], a2[1], h2, dis_col, b1.reshape(1, 128), x1,
                   linW, linb.reshape(1, 128))
    return out[:n]


def _counts_body(b_ref, v_ref, ratio_ref, k_ref):
    Np = b_ref.shape[1]
    G = k_ref.shape[0]
    gi = lax.broadcasted_iota(jnp.int32, (G, 1), 0).astype(jnp.float32)
    acc = jnp.zeros((G, 1), jnp.float32)
    BJ = 2048
    for c in range(Np // BJ):
        bj = b_ref[:, pl.ds(c * BJ, BJ)]
        vj = v_ref[:, pl.ds(c * BJ, BJ)]
        oh = jnp.where(bj == gi, vj, 0.0)
        acc = acc + jnp.sum(oh, axis=1, keepdims=True)
    k_ref[...] = jnp.ceil(ratio_ref[0, 0] * acc)


def _rank_body(sc_ref, bc_ref, vc_ref, sr_ref, br_ref, vr_ref,
               k_ref, koff_ref, tgt_ref, kept_ref, *, n_sentinel):
    BI = sc_ref.shape[0]
    Np = sr_ref.shape[1]
    G = k_ref.shape[1]
    i0 = pl.program_id(0) * BI
    si = sc_ref[...]
    bi = bc_ref[...]
    vi = vc_ref[...]
    ii = i0 + lax.broadcasted_iota(jnp.int32, (BI, 1), 0).astype(jnp.float32)
    acc = jnp.zeros((BI, 1), jnp.float32)
    BJ = 2048
    for c in range(Np // BJ):
        sj = sr_ref[:, pl.ds(c * BJ, BJ)]
        bj = br_ref[:, pl.ds(c * BJ, BJ)]
        vj = vr_ref[:, pl.ds(c * BJ, BJ)]
        jj = c * BJ + lax.broadcasted_iota(jnp.int32, (1, BJ), 1).astype(jnp.float32)
        beats = (sj > si) | ((sj == si) & (jj < ii))
        cmp = (bj == bi) & (vj > 0.0) & beats
        acc = acc + jnp.sum(cmp.astype(jnp.float32), axis=1, keepdims=True)
    ohm = bi == lax.broadcasted_iota(jnp.int32, (1, G), 1).astype(jnp.float32)
    kb = jnp.sum(jnp.where(ohm, k_ref[...], 0.0), axis=1, keepdims=True)
    kob = jnp.sum(jnp.where(ohm, koff_ref[...], 0.0), axis=1, keepdims=True)
    kept = (vi > 0.0) & (acc < kb)
    tgt_ref[...] = jnp.where(kept, kob + acc, float(n_sentinel))
    kept_ref[...] = kept.astype(jnp.float32)


def _topk_pool(x, edge_index, edge_mask, batch, node_mask, p, ratio, num_graphs):
    n, D = x.shape
    G = num_graphs
    Np = ((n + 2047) // 2048) * 2048
    bpad = jnp.zeros((Np, 1), jnp.float32).at[:n, 0].set(batch.astype(jnp.float32))
    vpad = jnp.zeros((Np, 1), jnp.float32).at[:n, 0].set(node_mask.astype(jnp.float32))

    scr = jnp.tanh((x @ p) / jnp.linalg.norm(p))
    scr_col = jnp.zeros((Np, 1), jnp.float32).at[:n, 0].set(scr)

    b_row = bpad.reshape(1, Np)
    v_row = vpad.reshape(1, Np)
    s_row = scr_col.reshape(1, Np)
    ratio_arr = jnp.full((1, 1), ratio, jnp.float32)

    k_col = pl.pallas_call(
        _counts_body,
        out_shape=jax.ShapeDtypeStruct((G, 1), jnp.float32),
    )(b_row, v_row, ratio_arr)
    k_row = k_col.reshape(1, G)
    koff_full = jnp.concatenate([jnp.zeros((1,), jnp.float32),
                                 jnp.cumsum(k_col[:, 0])])
    koff_row = koff_full[:G].reshape(1, G)
    total_kept = koff_full[G].astype(jnp.int32)

    BI = 256
    tgt_col, kept_col = pl.pallas_call(
        functools.partial(_rank_body, n_sentinel=n),
        grid=(Np // BI,),
        in_specs=[pl.BlockSpec((BI, 1), lambda i: (i, 0)),
                  pl.BlockSpec((BI, 1), lambda i: (i, 0)),
                  pl.BlockSpec((BI, 1), lambda i: (i, 0)),
                  pl.BlockSpec((1, Np), lambda i: (0, 0)),
                  pl.BlockSpec((1, Np), lambda i: (0, 0)),
                  pl.BlockSpec((1, Np), lambda i: (0, 0)),
                  pl.BlockSpec((1, G), lambda i: (0, 0)),
                  pl.BlockSpec((1, G), lambda i: (0, 0))],
        out_specs=[pl.BlockSpec((BI, 1), lambda i: (i, 0)),
                   pl.BlockSpec((BI, 1), lambda i: (i, 0))],
        out_shape=[jax.ShapeDtypeStruct((Np, 1), jnp.float32),
                   jax.ShapeDtypeStruct((Np, 1), jnp.float32)],
    )(scr_col, bpad, vpad, s_row, b_row, v_row, k_row, koff_row)

    tgt = tgt_col[:n, 0].astype(jnp.int32)
    kept = kept_col[:n, 0] > 0.0

    xn = jnp.zeros((n + 1, D), x.dtype).at[tgt].set(x * scr[:, None])[:n]
    bnew = jnp.zeros((n + 1,), batch.dtype).at[tgt].set(batch)[:n]
    nmask_new = jnp.arange(n, dtype=jnp.int32) < total_kept
    s, d = edge_index[0], edge_index[1]
    a = jnp.where(kept, tgt, -1)
    asv, adv = a[s], a[d]
    em = edge_mask & (asv >= 0) & (adv >= 0)
    src_new = jnp.where(em, asv, 0).astype(s.dtype)
    dst_new = jnp.where(em, adv, 0).astype(d.dtype)
    einew = jnp.stack([src_new, dst_new])
    return xn, einew, bnew, nmask_new, em


def _gadd(x, b, nmask, G):
    seg = jnp.where(nmask, b, G)
    return jnp.zeros((G + 1, x.shape[1]), x.dtype).at[seg].add(x)[:G]


def _gmax(x, b, nmask, G):
    seg = jnp.where(nmask, b, G)
    return jax.ops.segment_max(x, seg, num_segments=G + 1)[:G]


def _mlp_body(h_ref, g_ref, be_ref, w1_ref, b1_ref, w2_ref, b2_ref, o_ref):
    h = h_ref[...]
    h = (h / jnp.sqrt(1.0 + EPS)) * g_ref[...] + be_ref[...]
    h = jnp.maximum(jnp.dot(h, w1_ref[...], preferred_element_type=jnp.float32)
                    + b1_ref[...], 0.0)
    z = jnp.dot(h, w2_ref[...], preferred_element_type=jnp.float32) + b2_ref[...]
    z = z - jnp.max(z, axis=-1, keepdims=True)
    e = jnp.exp(z)
    o_ref[...] = e / jnp.sum(e, axis=-1, keepdims=True)


def _mlp_head(h, bn_gamma, bn_beta, lin1W, lin1b, lin2W, lin2b):
    G = h.shape[0]
    C = lin2W.shape[1]
    w2 = jnp.zeros((lin2W.shape[0], 128), jnp.float32).at[:, :C].set(lin2W)
    b2 = jnp.full((1, 128), -1e30, jnp.float32).at[0, :C].set(lin2b)
    out = pl.pallas_call(
        _mlp_body,
        out_shape=jax.ShapeDtypeStruct((G, 128), jnp.float32),
    )(h, bn_gamma[None, :], bn_beta[None, :], lin1W, lin1b[None, :], w2, b2)
    return out[:, :C]


def kernel(x, edge_index, batch,
           blk0_W0, blk0_b0, blk0_W1, blk0_b1, blk0_linW, blk0_linb,
           blk1_W0, blk1_b0, blk1_W1, blk1_b1, blk1_linW, blk1_linb,
           blk2_W0, blk2_b0, blk2_W1, blk2_b1, blk2_linW, blk2_linb,
           pool1_p, pool2_p, bn_gamma, bn_beta, lin1W, lin1b, lin2W, lin2b):
    P = dict(
        blk0_W0=blk0_W0, blk0_b0=blk0_b0, blk0_W1=blk0_W1, blk0_b1=blk0_b1,
        blk0_linW=blk0_linW, blk0_linb=blk0_linb,
        blk1_W0=blk1_W0, blk1_b0=blk1_b0, blk1_W1=blk1_W1, blk1_b1=blk1_b1,
        blk1_linW=blk1_linW, blk1_linb=blk1_linb,
        blk2_W0=blk2_W0, blk2_b0=blk2_b0, blk2_W1=blk2_W1, blk2_b1=blk2_b1,
        blk2_linW=blk2_linW, blk2_linb=blk2_linb,
        pool1_p=pool1_p, pool2_p=pool2_p,
    )
    G = N_GRAPHS
    n = x.shape[0]
    nmask = jnp.ones((n,), bool)
    emask = jnp.ones((edge_index.shape[1],), bool)
    x = _block(x, edge_index[0], edge_index[1], emask,
               P["blk0_W0"], P["blk0_b0"], P["blk0_W1"], P["blk0_b1"],
               P["blk0_linW"], P["blk0_linb"])
    xs = [_gadd(x, batch, nmask, G), _gmax(x, batch, nmask, G)]
    ei = edge_index
    for b in (1, 2):
        x, ei, batch, nmask, emask = _topk_pool(
            x, ei, emask, batch, nmask, P["pool%d_p" % b], RATIO, G)
        x = _block(x, ei[0], ei[1], emask,
                   P["blk%d_W0" % b], P["blk%d_b0" % b],
                   P["blk%d_W1" % b], P["blk%d_b1" % b],
                   P["blk%d_linW" % b], P["blk%d_linb" % b])
        xs.extend([_gadd(x, batch, nmask, G), _gmax(x, batch, nmask, G)])
    h = jnp.concatenate(xs, axis=1)
    return _mlp_head(h, bn_gamma, bn_beta, lin1W, lin1b, lin2W, lin2b)
